# baseline (speedup 1.0000x reference)
import jax
import jax.numpy as jnp
import numpy as np
from jax import lax
from jax.experimental import pallas as pl
from jax.experimental.pallas import tpu as pltpu

_N = 10000
_DIM = 128
_H = 8
_DH = 16
_E = 320000
_BLK = 200
_G = _N // _BLK
_EBLK = 4000
_EG = _E // _EBLK


def _dense_pre(x, W_gat, skip_W, P, skip_b):
    def body(x_ref, wg_ref, sw_ref, p_ref, sb_ref,
             h_ref, idn_ref, as_ref, adst_ref, c_ref, mx_ref):
        i = pl.program_id(0)
        xb = x_ref[...]
        h = jnp.dot(xb, wg_ref[...], preferred_element_type=jnp.float32)
        h_ref[...] = h
        idn_ref[...] = (
            jnp.dot(xb, sw_ref[...], preferred_element_type=jnp.float32)
            + sb_ref[...])
        ad = jnp.dot(h, p_ref[...], preferred_element_type=jnp.float32)
        asb = ad[:, 0:16]
        adb = ad[:, 16:32]
        as_ref[...] = asb
        adst_ref[...] = adb
        ms = jnp.max(asb, axis=0, keepdims=True)
        md = jnp.max(adb, axis=0, keepdims=True)
        m = jnp.concatenate([ms, md], axis=1)

        @pl.when(i == 0)
        def _():
            mx_ref[...] = jnp.broadcast_to(m, (8, 32))

        @pl.when(i > 0)
        def _():
            mx_ref[...] = jnp.maximum(mx_ref[...], m)

        @pl.when(i == _G - 1)
        def _():
            s = mx_ref[:, 0:16] + mx_ref[:, 16:32]
            c_ref[...] = jnp.where(s > 0, s, 0.2 * s)

    return pl.pallas_call(
        body,
        grid=(_G,),
        in_specs=[
            pl.BlockSpec((_BLK, _DIM), lambda i: (i, 0)),
            pl.BlockSpec((_DIM, _DIM), lambda i: (0, 0)),
            pl.BlockSpec((_DIM, _DIM), lambda i: (0, 0)),
            pl.BlockSpec((_DIM, 32), lambda i: (0, 0)),
            pl.BlockSpec((1, _DIM), lambda i: (0, 0)),
        ],
        out_specs=[
            pl.BlockSpec((_BLK, _DIM), lambda i: (i, 0)),
            pl.BlockSpec((_BLK, _DIM), lambda i: (i, 0)),
            pl.BlockSpec((_BLK, 16), lambda i: (i, 0)),
            pl.BlockSpec((_BLK, 16), lambda i: (i, 0)),
            pl.BlockSpec((8, 16), lambda i: (0, 0)),
        ],
        out_shape=[
            jax.ShapeDtypeStruct((_N, _DIM), jnp.float32),
            jax.ShapeDtypeStruct((_N, _DIM), jnp.float32),
            jax.ShapeDtypeStruct((_N, 16), jnp.float32),
            jax.ShapeDtypeStruct((_N, 16), jnp.float32),
            jax.ShapeDtypeStruct((8, 16), jnp.float32),
        ],
        scratch_shapes=[pltpu.VMEM((8, 32), jnp.float32)],
    )(x, W_gat, skip_W, P, skip_b)


def _edge_w(e_src, e_dst, cmat):
    def body(es_ref, ed_ref, c_ref, w_ref):
        e = es_ref[...] + ed_ref[...]
        e = jnp.where(e > 0, e, 0.2 * e)
        w_ref[...] = jnp.exp(e - c_ref[...])

    return pl.pallas_call(
        body,
        grid=(_EG,),
        in_specs=[
            pl.BlockSpec((_EBLK, 8), lambda i: (i, 0)),
            pl.BlockSpec((_EBLK, 8), lambda i: (i, 0)),
            pl.BlockSpec((1, 8), lambda i: (0, 0)),
        ],
        out_specs=pl.BlockSpec((_EBLK, 8), lambda i: (i, 0)),
        out_shape=jax.ShapeDtypeStruct((_E, 8), jnp.float32),
    )(e_src, e_dst, cmat)


def _merge(op, dp, emat, bias):
    def body(op_ref, dp_ref, e_ref, b_ref, pre_ref, s1_ref, s2_ref,
             acc1, acc2):
        i = pl.program_id(0)
        o = op_ref[...]
        den = dp_ref[...]
        den128 = jnp.dot(den, e_ref[...], preferred_element_type=jnp.float32)
        pre = o / (den128 + 1e-16) + b_ref[...]
        pre_ref[...] = pre
        cs = jnp.sum(pre, axis=0, keepdims=True)
        cq = jnp.sum(pre * pre, axis=0, keepdims=True)

        @pl.when(i == 0)
        def _():
            acc1[...] = cs
            acc2[...] = cq

        @pl.when(i > 0)
        def _():
            acc1[...] = acc1[...] + cs
            acc2[...] = acc2[...] + cq

        @pl.when(i == _G - 1)
        def _():
            s1_ref[...] = acc1[...]
            s2_ref[...] = acc2[...]

    return pl.pallas_call(
        body,
        grid=(_G,),
        in_specs=[
            pl.BlockSpec((_BLK, _DIM), lambda i: (i, 0)),
            pl.BlockSpec((_BLK, 16), lambda i: (i, 0)),
            pl.BlockSpec((16, _DIM), lambda i: (0, 0)),
            pl.BlockSpec((1, _DIM), lambda i: (0, 0)),
        ],
        out_specs=[
            pl.BlockSpec((_BLK, _DIM), lambda i: (i, 0)),
            pl.BlockSpec((1, _DIM), lambda i: (0, 0)),
            pl.BlockSpec((1, _DIM), lambda i: (0, 0)),
        ],
        out_shape=[
            jax.ShapeDtypeStruct((_N, _DIM), jnp.float32),
            jax.ShapeDtypeStruct((1, _DIM), jnp.float32),
            jax.ShapeDtypeStruct((1, _DIM), jnp.float32),
        ],
        scratch_shapes=[
            pltpu.VMEM((1, _DIM), jnp.float32),
            pltpu.VMEM((1, _DIM), jnp.float32),
        ],
    )(op, dp, emat, bias)


def _finish(pre, s1, s2, gamma, beta, idn):
    inv_n = 1.0 / _N

    def body(pre_ref, s1_ref, s2_ref, g_ref, b_ref, idn_ref, o_ref):
        mean = s1_ref[...] * inv_n
        var = s2_ref[...] * inv_n - mean * mean
        y = (pre_ref[...] - mean) * lax.rsqrt(var + 1e-5) * g_ref[...] + b_ref[...]
        gl = 0.5 * y * (1.0 + lax.erf(y * 0.7071067811865476))
        o_ref[...] = gl + idn_ref[...]

    return pl.pallas_call(
        body,
        grid=(_G,),
        in_specs=[
            pl.BlockSpec((_BLK, _DIM), lambda i: (i, 0)),
            pl.BlockSpec((1, _DIM), lambda i: (0, 0)),
            pl.BlockSpec((1, _DIM), lambda i: (0, 0)),
            pl.BlockSpec((1, _DIM), lambda i: (0, 0)),
            pl.BlockSpec((1, _DIM), lambda i: (0, 0)),
            pl.BlockSpec((_BLK, _DIM), lambda i: (i, 0)),
        ],
        out_specs=pl.BlockSpec((_BLK, _DIM), lambda i: (i, 0)),
        out_shape=jax.ShapeDtypeStruct((_N, _DIM), jnp.float32),
    )(pre, s1, s2, gamma, beta, idn)


def kernel(x, edge_index, W_gat, att_src, att_dst, bias_gat,
           bn_gamma, bn_beta, skip_W, skip_b):
    rows = np.arange(_DIM)
    hh = rows // _DH
    P = jnp.zeros((_DIM, 32), jnp.float32)
    asf = att_src.reshape(_DIM)
    adf = att_dst.reshape(_DIM)
    P = P.at[rows, hh].set(asf)
    P = P.at[rows, hh + 8].set(asf)
    P = P.at[rows, 16 + hh].set(adf)
    P = P.at[rows, 24 + hh].set(adf)

    emat = np.zeros((16, _DIM), np.float32)
    emat[hh, rows] = 1.0
    emat = jnp.asarray(emat)

    src = edge_index[0].astype(jnp.int32)
    dst = edge_index[1].astype(jnp.int32)

    h, idn, astab, adtab, cmat = _dense_pre(
        x, W_gat, skip_W, P, skip_b.reshape(1, _DIM))

    w = _edge_w(astab[src, :8], adtab[dst, :8], cmat[0:1, 0:8])
    dp = jax.ops.segment_sum(w, dst, num_segments=_N)
    dp = jnp.pad(dp, ((0, 0), (0, 8)))
    msg = h.reshape(_N, _H, _DH)[src] * w[:, :, None]
    op = jax.ops.segment_sum(msg, dst, num_segments=_N).reshape(_N, _DIM)

    pre, s1, s2 = _merge(op, dp, emat, bias_gat.reshape(1, _DIM))
    return _finish(pre, s1, s2, bn_gamma.reshape(1, _DIM),
                   bn_beta.reshape(1, _DIM), idn)


# speedup vs baseline: 1.0003x; 1.0003x over previous
import jax
import jax.numpy as jnp
import numpy as np
from jax import lax
from jax.experimental import pallas as pl
from jax.experimental.pallas import tpu as pltpu

_N = 10000
_DIM = 128
_H = 8
_DH = 16
_E = 320000
_BLK = 200
_G = _N // _BLK
_EBLK = 4000
_EG = _E // _EBLK


def _dense_pre(x, W_gat, skip_W, P, skip_b):
    def body(x_ref, wg_ref, sw_ref, p_ref, sb_ref,
             h_ref, idn_ref, as_ref, adst_ref, c_ref, mx_ref):
        i = pl.program_id(0)
        xb = x_ref[...]
        h = jnp.dot(xb, wg_ref[...], preferred_element_type=jnp.float32)
        h_ref[...] = h
        idn_ref[...] = (
            jnp.dot(xb, sw_ref[...], preferred_element_type=jnp.float32)
            + sb_ref[...])
        ad = jnp.dot(h, p_ref[...], preferred_element_type=jnp.float32)
        asb = ad[:, 0:16]
        adb = ad[:, 16:32]
        as_ref[...] = asb
        adst_ref[...] = adb
        ms = jnp.max(asb, axis=0, keepdims=True)
        md = jnp.max(adb, axis=0, keepdims=True)
        m = jnp.concatenate([ms, md], axis=1)

        @pl.when(i == 0)
        def _():
            mx_ref[...] = jnp.broadcast_to(m, (8, 32))

        @pl.when(i > 0)
        def _():
            mx_ref[...] = jnp.maximum(mx_ref[...], m)

        @pl.when(i == _G - 1)
        def _():
            s = mx_ref[:, 0:16] + mx_ref[:, 16:32]
            c_ref[...] = jnp.where(s > 0, s, 0.2 * s)

    return pl.pallas_call(
        body,
        grid=(_G,),
        in_specs=[
            pl.BlockSpec((_BLK, _DIM), lambda i: (i, 0)),
            pl.BlockSpec((_DIM, _DIM), lambda i: (0, 0)),
            pl.BlockSpec((_DIM, _DIM), lambda i: (0, 0)),
            pl.BlockSpec((_DIM, 32), lambda i: (0, 0)),
            pl.BlockSpec((1, _DIM), lambda i: (0, 0)),
        ],
        out_specs=[
            pl.BlockSpec((_BLK, _DIM), lambda i: (i, 0)),
            pl.BlockSpec((_BLK, _DIM), lambda i: (i, 0)),
            pl.BlockSpec((_BLK, 16), lambda i: (i, 0)),
            pl.BlockSpec((_BLK, 16), lambda i: (i, 0)),
            pl.BlockSpec((8, 16), lambda i: (0, 0)),
        ],
        out_shape=[
            jax.ShapeDtypeStruct((_N, _DIM), jnp.float32),
            jax.ShapeDtypeStruct((_N, _DIM), jnp.float32),
            jax.ShapeDtypeStruct((_N, 16), jnp.float32),
            jax.ShapeDtypeStruct((_N, 16), jnp.float32),
            jax.ShapeDtypeStruct((8, 16), jnp.float32),
        ],
        scratch_shapes=[pltpu.VMEM((8, 32), jnp.float32)],
    )(x, W_gat, skip_W, P, skip_b)


def _edge_w(e_src, e_dst, cmat):
    def body(es_ref, ed_ref, c_ref, w_ref):
        e = es_ref[...] + ed_ref[...]
        e = jnp.where(e > 0, e, 0.2 * e)
        w_ref[...] = jnp.exp(e - c_ref[...])

    return pl.pallas_call(
        body,
        grid=(_EG,),
        in_specs=[
            pl.BlockSpec((_EBLK, 8), lambda i: (i, 0)),
            pl.BlockSpec((_EBLK, 8), lambda i: (i, 0)),
            pl.BlockSpec((1, 8), lambda i: (0, 0)),
        ],
        out_specs=pl.BlockSpec((_EBLK, 8), lambda i: (i, 0)),
        out_shape=jax.ShapeDtypeStruct((_E, 8), jnp.float32),
    )(e_src, e_dst, cmat)


def _merge(op, dp, emat, bias):
    def body(op_ref, dp_ref, e_ref, b_ref, pre_ref, s1_ref, s2_ref,
             acc1, acc2):
        i = pl.program_id(0)
        o = op_ref[...]
        den = dp_ref[...]
        den128 = jnp.dot(den, e_ref[...], preferred_element_type=jnp.float32)
        pre = o / (den128 + 1e-16) + b_ref[...]
        pre_ref[...] = pre
        cs = jnp.sum(pre, axis=0, keepdims=True)
        cq = jnp.sum(pre * pre, axis=0, keepdims=True)

        @pl.when(i == 0)
        def _():
            acc1[...] = cs
            acc2[...] = cq

        @pl.when(i > 0)
        def _():
            acc1[...] = acc1[...] + cs
            acc2[...] = acc2[...] + cq

        @pl.when(i == _G - 1)
        def _():
            s1_ref[...] = acc1[...]
            s2_ref[...] = acc2[...]

    return pl.pallas_call(
        body,
        grid=(_G,),
        in_specs=[
            pl.BlockSpec((_BLK, _DIM), lambda i: (i, 0)),
            pl.BlockSpec((_BLK, 16), lambda i: (i, 0)),
            pl.BlockSpec((16, _DIM), lambda i: (0, 0)),
            pl.BlockSpec((1, _DIM), lambda i: (0, 0)),
        ],
        out_specs=[
            pl.BlockSpec((_BLK, _DIM), lambda i: (i, 0)),
            pl.BlockSpec((1, _DIM), lambda i: (0, 0)),
            pl.BlockSpec((1, _DIM), lambda i: (0, 0)),
        ],
        out_shape=[
            jax.ShapeDtypeStruct((_N, _DIM), jnp.float32),
            jax.ShapeDtypeStruct((1, _DIM), jnp.float32),
            jax.ShapeDtypeStruct((1, _DIM), jnp.float32),
        ],
        scratch_shapes=[
            pltpu.VMEM((1, _DIM), jnp.float32),
            pltpu.VMEM((1, _DIM), jnp.float32),
        ],
    )(op, dp, emat, bias)


def _finish(pre, s1, s2, gamma, beta, idn):
    inv_n = 1.0 / _N

    def body(pre_ref, s1_ref, s2_ref, g_ref, b_ref, idn_ref, o_ref):
        mean = s1_ref[...] * inv_n
        var = s2_ref[...] * inv_n - mean * mean
        y = (pre_ref[...] - mean) * lax.rsqrt(var + 1e-5) * g_ref[...] + b_ref[...]
        gl = 0.5 * y * (1.0 + lax.erf(y * 0.7071067811865476))
        o_ref[...] = gl + idn_ref[...]

    return pl.pallas_call(
        body,
        grid=(_G,),
        in_specs=[
            pl.BlockSpec((_BLK, _DIM), lambda i: (i, 0)),
            pl.BlockSpec((1, _DIM), lambda i: (0, 0)),
            pl.BlockSpec((1, _DIM), lambda i: (0, 0)),
            pl.BlockSpec((1, _DIM), lambda i: (0, 0)),
            pl.BlockSpec((1, _DIM), lambda i: (0, 0)),
            pl.BlockSpec((_BLK, _DIM), lambda i: (i, 0)),
        ],
        out_specs=pl.BlockSpec((_BLK, _DIM), lambda i: (i, 0)),
        out_shape=jax.ShapeDtypeStruct((_N, _DIM), jnp.float32),
    )(pre, s1, s2, gamma, beta, idn)


def kernel(x, edge_index, W_gat, att_src, att_dst, bias_gat,
           bn_gamma, bn_beta, skip_W, skip_b):
    rows = np.arange(_DIM)
    hh = rows // _DH
    P = jnp.zeros((_DIM, 32), jnp.float32)
    asf = att_src.reshape(_DIM)
    adf = att_dst.reshape(_DIM)
    P = P.at[rows, hh].set(asf)
    P = P.at[rows, hh + 8].set(asf)
    P = P.at[rows, 16 + hh].set(adf)
    P = P.at[rows, 24 + hh].set(adf)

    emat = np.zeros((16, _DIM), np.float32)
    emat[hh, rows] = 1.0
    emat = jnp.asarray(emat)

    src = edge_index[0].astype(jnp.int32)
    dst = edge_index[1].astype(jnp.int32)

    h, idn, astab, adtab, cmat = _dense_pre(
        x, W_gat, skip_W, P, skip_b.reshape(1, _DIM))

    e = astab[src, :8] + adtab[dst, :8]
    e = jnp.where(e > 0, e, 0.2 * e)
    w = jnp.exp(e - cmat[0, :8][None, :])
    dp = jax.ops.segment_sum(w, dst, num_segments=_N)
    dp = jnp.pad(dp, ((0, 0), (0, 8)))
    msg = h.reshape(_N, _H, _DH)[src] * w[:, :, None]
    op = jax.ops.segment_sum(msg, dst, num_segments=_N).reshape(_N, _DIM)

    pre, s1, s2 = _merge(op, dp, emat, bias_gat.reshape(1, _DIM))
    return _finish(pre, s1, s2, bn_gamma.reshape(1, _DIM),
                   bn_beta.reshape(1, _DIM), idn)
